# Initial kernel scaffold; baseline (speedup 1.0000x reference)
import jax
import jax.numpy as jnp
from jax.experimental import pallas as pl


def kernel(inputs, table):
    raise NotImplementedError("write your pallas kernel here")


# speedup vs baseline: 3.1344x; 3.1344x over previous
import functools

import jax
import jax.numpy as jnp
from jax import lax
from jax.experimental import pallas as pl
from jax.experimental.pallas import tpu as pltpu
from jax.experimental.pallas import tpu_sc as plsc

B, L, C = 4096, 200, 128
VOCAB = 262
HALF = C // 2

NC, NS, LANES = 2, 16, 16
NW = NC * NS
BL = B * L
PER_W = BL // NW
CHUNK = 128
NCHUNK = PER_W // CHUNK

VPAD = 512
ROWS_W = VPAD // NW

_MESH = plsc.VectorSubcoreMesh(core_axis_name="c", subcore_axis_name="s")


@functools.partial(
    pl.kernel,
    out_type=jax.ShapeDtypeStruct((VPAD, HALF), jnp.float32),
    mesh=_MESH,
    scratch_types=[
        pltpu.VMEM((ROWS_W * C,), jnp.float32),
        pltpu.VMEM((ROWS_W, HALF), jnp.float32),
    ],
    compiler_params=pltpu.CompilerParams(needs_layout_passes=False),
)
def _pool_table(tab_hbm, ptab_hbm, tab_v, ptab_v):
    wid = lax.axis_index("s") * NC + lax.axis_index("c")
    base = wid * ROWS_W
    for r in range(ROWS_W):
        pltpu.sync_copy(tab_hbm.at[base + r], tab_v.at[pl.ds(r * C, C)])
    lanes = lax.iota(jnp.int32, LANES)
    for r in range(ROWS_W):
        for j in range(HALF // LANES):
            flat = r * C + (j * LANES + lanes) * 2
            even = plsc.load_gather(tab_v, [flat])
            odd = plsc.load_gather(tab_v, [flat + 1])
            ptab_v[r, pl.ds(j * LANES, LANES)] = jnp.maximum(even, odd)
    pltpu.sync_copy(ptab_v, ptab_hbm.at[pl.ds(base, ROWS_W)])


@functools.partial(
    pl.kernel,
    out_type=(
        jax.ShapeDtypeStruct((BL, C), jnp.float32),
        jax.ShapeDtypeStruct((BL, HALF), jnp.float32),
    ),
    mesh=_MESH,
    scratch_types=[
        pltpu.VMEM((PER_W,), jnp.int32),
        pltpu.VMEM((CHUNK, C), jnp.float32),
        pltpu.VMEM((CHUNK, HALF), jnp.float32),
        pltpu.SemaphoreType.DMA,
        pltpu.SemaphoreType.DMA,
    ],
    compiler_params=pltpu.CompilerParams(use_tc_tiling_on_sc=False),
)
def _gather(tab_hbm, ptab_hbm, idx_hbm, outx_hbm, outp_hbm,
            idx_v, x_v, p_v, sem_x, sem_p):
    wid = lax.axis_index("s") * NC + lax.axis_index("c")
    base = wid * PER_W
    pltpu.sync_copy(idx_hbm.at[pl.ds(base, PER_W)], idx_v)

    def chunk(i, carry):
        row0 = base + i * CHUNK
        ids = idx_v.at[pl.ds(i * CHUNK, CHUNK)]
        cx = pltpu.async_copy(tab_hbm.at[ids], x_v, sem_x)
        cp = pltpu.async_copy(ptab_hbm.at[ids], p_v, sem_p)
        cx.wait()
        cp.wait()
        pltpu.sync_copy(x_v, outx_hbm.at[pl.ds(row0, CHUNK)])
        pltpu.sync_copy(p_v, outp_hbm.at[pl.ds(row0, CHUNK)])
        return carry

    lax.fori_loop(0, NCHUNK, chunk, 0)


def kernel(inputs, table):
    idx_flat = inputs.astype(jnp.int32).reshape(BL)
    tab_pad = jnp.pad(table, ((0, VPAD - VOCAB), (0, 0)))
    ptab = _pool_table(tab_pad)
    x_flat, p_flat = _gather(table, ptab, idx_flat)
    return x_flat.reshape(B, L, C), p_flat.reshape(B, L, HALF)
